# baseline (speedup 1.0000x reference)
import functools

import jax
import jax.numpy as jnp
from jax import lax
from jax.experimental import pallas as pl
from jax.experimental.pallas import tpu as pltpu
from jax.experimental.pallas import tpu_sc as plsc

D = 64
NC = 2
NS = 16
NW = NC * NS
CHUNK = 128


@functools.partial(jax.jit, static_argnums=(0,))
def _router_sc(B, first2d, token_emb, wb):
    bpw = B // NW
    nchunk = bpw // CHUNK
    ngroups = bpw // 16

    mesh = plsc.VectorSubcoreMesh(core_axis_name="c", subcore_axis_name="s")

    @functools.partial(
        pl.kernel,
        mesh=mesh,
        compiler_params=pltpu.CompilerParams(
            needs_layout_passes=False,
            use_tc_tiling_on_sc=False,
        ),
        out_type=jax.ShapeDtypeStruct((B * 2,), jnp.float32),
        scratch_types=[
            pltpu.VMEM((nchunk, CHUNK), jnp.int32),
            pltpu.VMEM((bpw, D), jnp.float32),
            pltpu.VMEM((bpw * 2,), jnp.float32),
            pltpu.VMEM((144,), jnp.float32),
            pltpu.SemaphoreType.DMA,
        ],
    )
    def k(first_hbm, table_hbm, wb_hbm, out_hbm, idx_v, rows_v, out_v, wb_v, sem):
        wid = lax.axis_index("s") * NC + lax.axis_index("c")
        base = wid * bpw

        pltpu.sync_copy(wb_hbm, wb_v)
        pltpu.sync_copy(first_hbm.at[pl.ds(wid * nchunk, nchunk)], idx_v)

        copies = []
        for j in range(nchunk):
            copies.append(
                pltpu.async_copy(
                    table_hbm.at[idx_v.at[j]],
                    rows_v.at[pl.ds(j * CHUNK, CHUNK)],
                    sem,
                )
            )
        for c in copies:
            c.wait()

        iota16 = lax.iota(jnp.int32, 16)

        w0 = [wb_v[pl.ds(16 * c, 16)] for c in range(D // 16)]
        w1 = [wb_v[pl.ds(D + 16 * c, 16)] for c in range(D // 16)]
        bias = wb_v[pl.ds(2 * D, 16)]

        def g_body(g, carry):
            row_idx = iota16 + g * 16
            a0 = jnp.zeros((16,), jnp.float32)
            a1 = jnp.zeros((16,), jnp.float32)
            for d in range(D):
                col_idx = jnp.full((16,), d, jnp.int32)
                col = plsc.load_gather(rows_v, [row_idx, col_idx])
                a0 = a0 + col * w0[d // 16][d % 16]
                a1 = a1 + col * w1[d // 16][d % 16]
            a0 = a0 + bias[0]
            a1 = a1 + bias[1]
            out_pos = row_idx * 2
            plsc.store_scatter(out_v, [out_pos], a0)
            plsc.store_scatter(out_v, [out_pos + 1], a1)
            return carry

        lax.fori_loop(0, ngroups, g_body, 0)

        pltpu.sync_copy(out_v, out_hbm.at[pl.ds(base * 2, bpw * 2)])

    return k(first2d, token_emb, wb)


def kernel(ids, token_emb, fc_w, fc_b):
    B = ids.shape[0]
    first2d = ids[:, 0].reshape(B // CHUNK, CHUNK).astype(jnp.int32)
    wb = jnp.zeros((144,), jnp.float32)
    wb = wb.at[: 2 * D].set(fc_w.reshape(-1))
    wb = wb.at[2 * D : 2 * D + 2].set(fc_b)
    out = _router_sc(B, first2d, token_emb, wb)
    return out.reshape(B, 2)


# speedup vs baseline: 3.9596x; 3.9596x over previous
import functools

import jax
import jax.numpy as jnp
from jax import lax
from jax.experimental import pallas as pl
from jax.experimental.pallas import tpu as pltpu
from jax.experimental.pallas import tpu_sc as plsc

D = 64
NC = 2
NS = 16
NW = NC * NS
CH = 128
COLS = 8192


def _proj_body(tT_ref, w_ref, o0_ref, o1_ref):
    x = tT_ref[...]
    w = w_ref[...]
    p = jnp.dot(w, x, preferred_element_type=jnp.float32)
    o0_ref[...] = p[0]
    o1_ref[...] = p[1]


@functools.partial(jax.jit, static_argnums=(0, 1))
def _router(B, V, first, token_embT, fc_w, fc_b):
    ncols = (V + COLS - 1) // COLS
    p0, p1 = pl.pallas_call(
        _proj_body,
        grid=(ncols,),
        in_specs=[
            pl.BlockSpec((D, COLS), lambda c: (0, c)),
            pl.BlockSpec((2, D), lambda c: (0, 0)),
        ],
        out_specs=[
            pl.BlockSpec((COLS,), lambda c: (c,)),
            pl.BlockSpec((COLS,), lambda c: (c,)),
        ],
        out_shape=[
            jax.ShapeDtypeStruct((V,), jnp.float32),
            jax.ShapeDtypeStruct((V,), jnp.float32),
        ],
    )(token_embT, fc_w)

    bpw = B // NW
    nchunk = bpw // CH
    mesh = plsc.VectorSubcoreMesh(core_axis_name="c", subcore_axis_name="s")

    @functools.partial(
        pl.kernel,
        mesh=mesh,
        compiler_params=pltpu.CompilerParams(needs_layout_passes=False),
        out_type=jax.ShapeDtypeStruct((B * 2,), jnp.float32),
        scratch_types=[
            pltpu.VMEM((bpw,), jnp.int32),
            pltpu.VMEM((bpw,), jnp.float32),
            pltpu.VMEM((bpw,), jnp.float32),
            pltpu.VMEM((bpw * 2,), jnp.float32),
            pltpu.VMEM((16,), jnp.float32),
            pltpu.SemaphoreType.DMA,
        ],
    )
    def gather_k(p0_hbm, p1_hbm, first_hbm, b_hbm, out_hbm,
                 idsv, g0, g1, outv, bv, sem):
        wid = lax.axis_index("s") * NC + lax.axis_index("c")
        base = wid * bpw

        pltpu.sync_copy(b_hbm, bv)
        pltpu.sync_copy(first_hbm.at[pl.ds(base, bpw)], idsv)

        copies = []
        for j in range(nchunk):
            copies.append(pltpu.async_copy(
                p0_hbm.at[idsv.at[pl.ds(j * CH, CH)]],
                g0.at[pl.ds(j * CH, CH)], sem))
            copies.append(pltpu.async_copy(
                p1_hbm.at[idsv.at[pl.ds(j * CH, CH)]],
                g1.at[pl.ds(j * CH, CH)], sem))
        for c in copies:
            c.wait()

        i16 = lax.iota(jnp.int32, 16)
        bvec = bv[pl.ds(0, 16)]
        b0 = bvec[0]
        b1 = bvec[1]

        def body(g, carry):
            v0 = g0[pl.ds(g * 16, 16)] + b0
            v1 = g1[pl.ds(g * 16, 16)] + b1
            pos = (i16 + g * 16) * 2
            plsc.store_scatter(outv, [pos], v0)
            plsc.store_scatter(outv, [pos + 1], v1)
            return carry

        lax.fori_loop(0, bpw // 16, body, 0)
        pltpu.sync_copy(outv, out_hbm.at[pl.ds(base * 2, bpw * 2)])

    bias16 = jnp.zeros((16,), jnp.float32).at[:2].set(fc_b)
    return gather_k(p0, p1, first, bias16)


def kernel(ids, token_emb, fc_w, fc_b):
    B = ids.shape[0]
    V = token_emb.shape[0]
    first = ids[:, 0].astype(jnp.int32)
    token_embT = token_emb.T
    out = _router(B, V, first, token_embT, fc_w, fc_b)
    return out.reshape(B, 2)


# speedup vs baseline: 6.3675x; 1.6081x over previous
import functools

import jax
import jax.numpy as jnp
from jax import lax
from jax.experimental import pallas as pl
from jax.experimental.pallas import tpu as pltpu
from jax.experimental.pallas import tpu_sc as plsc

D = 64
NC = 2
NS = 16
NW = NC * NS
CH = 128
COLS = 32768


def _proj_body(tT_ref, w_ref, o0_ref, o1_ref):
    x = tT_ref[...]
    w = w_ref[...]
    p = jnp.dot(w, x, preferred_element_type=jnp.float32)
    o0_ref[...] = p[0]
    o1_ref[...] = p[1]


@functools.partial(jax.jit, static_argnums=(0, 1))
def _router(B, V, first, token_embT, fc_w, fc_b):
    ncols = (V + COLS - 1) // COLS
    p0, p1 = pl.pallas_call(
        _proj_body,
        grid=(ncols,),
        in_specs=[
            pl.BlockSpec((D, COLS), lambda c: (0, c)),
            pl.BlockSpec((2, D), lambda c: (0, 0)),
        ],
        out_specs=[
            pl.BlockSpec((COLS,), lambda c: (c,)),
            pl.BlockSpec((COLS,), lambda c: (c,)),
        ],
        out_shape=[
            jax.ShapeDtypeStruct((V,), jnp.float32),
            jax.ShapeDtypeStruct((V,), jnp.float32),
        ],
    )(token_embT, fc_w)

    bpw = B // NW
    nchunk = bpw // CH
    mesh = plsc.VectorSubcoreMesh(core_axis_name="c", subcore_axis_name="s")

    @functools.partial(
        pl.kernel,
        mesh=mesh,
        compiler_params=pltpu.CompilerParams(needs_layout_passes=False),
        out_type=jax.ShapeDtypeStruct((2, B), jnp.float32),
        scratch_types=[
            pltpu.VMEM((bpw,), jnp.int32),
            pltpu.VMEM((bpw,), jnp.float32),
            pltpu.VMEM((bpw,), jnp.float32),
            pltpu.VMEM((16,), jnp.float32),
            pltpu.SemaphoreType.DMA,
        ],
    )
    def gather_k(p0_hbm, p1_hbm, first_hbm, b_hbm, out_hbm,
                 idsv, g0, g1, bv, sem):
        wid = lax.axis_index("s") * NC + lax.axis_index("c")
        base = wid * bpw

        pltpu.sync_copy(b_hbm, bv)
        pltpu.sync_copy(first_hbm.at[pl.ds(base, bpw)], idsv)

        copies = []
        for j in range(nchunk):
            copies.append(pltpu.async_copy(
                p0_hbm.at[idsv.at[pl.ds(j * CH, CH)]],
                g0.at[pl.ds(j * CH, CH)], sem))
            copies.append(pltpu.async_copy(
                p1_hbm.at[idsv.at[pl.ds(j * CH, CH)]],
                g1.at[pl.ds(j * CH, CH)], sem))
        for c in copies:
            c.wait()

        bvec = bv[pl.ds(0, 16)]
        b0 = bvec[0]
        b1 = bvec[1]

        def body(g, carry):
            g0[pl.ds(g * 16, 16)] = g0[pl.ds(g * 16, 16)] + b0
            g1[pl.ds(g * 16, 16)] = g1[pl.ds(g * 16, 16)] + b1
            return carry

        lax.fori_loop(0, bpw // 16, body, 0)
        pltpu.sync_copy(g0, out_hbm.at[0, pl.ds(base, bpw)])
        pltpu.sync_copy(g1, out_hbm.at[1, pl.ds(base, bpw)])

    bias16 = jnp.zeros((16,), jnp.float32).at[:2].set(fc_b)
    return gather_k(p0, p1, first, bias16)


def kernel(ids, token_emb, fc_w, fc_b):
    B = ids.shape[0]
    V = token_emb.shape[0]
    first = ids[:, 0].astype(jnp.int32)
    token_embT = token_emb.T
    out2 = _router(B, V, first, token_embT, fc_w, fc_b)
    return out2.T


# speedup vs baseline: 6.5157x; 1.0233x over previous
import functools

import jax
import jax.numpy as jnp
from jax import lax
from jax.experimental import pallas as pl
from jax.experimental.pallas import tpu as pltpu
from jax.experimental.pallas import tpu_sc as plsc

D = 64
NC = 2
NS = 16
NW = NC * NS
CH = 128
COLS = 65536


def _proj_body(tT_ref, w_ref, b_ref, o0_ref, o1_ref):
    x = tT_ref[...]
    w = w_ref[...]
    p = jnp.dot(w, x, preferred_element_type=jnp.float32)
    p = p + b_ref[...]
    o0_ref[...] = p[0]
    o1_ref[...] = p[1]


@functools.partial(jax.jit, static_argnums=(0, 1))
def _router(B, V, idsT, token_embT, fc_w, fc_b2):
    ncols = (V + COLS - 1) // COLS
    p0, p1 = pl.pallas_call(
        _proj_body,
        grid=(ncols,),
        in_specs=[
            pl.BlockSpec((D, COLS), lambda c: (0, c)),
            pl.BlockSpec((2, D), lambda c: (0, 0)),
            pl.BlockSpec((2, 1), lambda c: (0, 0)),
        ],
        out_specs=[
            pl.BlockSpec((COLS,), lambda c: (c,)),
            pl.BlockSpec((COLS,), lambda c: (c,)),
        ],
        out_shape=[
            jax.ShapeDtypeStruct((V,), jnp.float32),
            jax.ShapeDtypeStruct((V,), jnp.float32),
        ],
    )(token_embT, fc_w, fc_b2)

    bpw = B // NW
    nchunk = bpw // CH
    mesh = plsc.VectorSubcoreMesh(core_axis_name="c", subcore_axis_name="s")

    @functools.partial(
        pl.kernel,
        mesh=mesh,
        compiler_params=pltpu.CompilerParams(needs_layout_passes=False),
        out_type=jax.ShapeDtypeStruct((2, B), jnp.float32),
        scratch_types=[
            pltpu.VMEM((bpw,), jnp.int32),
            pltpu.VMEM((bpw,), jnp.float32),
            pltpu.VMEM((bpw,), jnp.float32),
            pltpu.SemaphoreType.DMA,
        ],
    )
    def gather_k(p0_hbm, p1_hbm, idsT_hbm, out_hbm, idsv, g0, g1, sem):
        wid = lax.axis_index("s") * NC + lax.axis_index("c")
        base = wid * bpw

        pltpu.sync_copy(idsT_hbm.at[0, pl.ds(base, bpw)], idsv)

        copies = []
        for j in range(nchunk):
            copies.append(pltpu.async_copy(
                p0_hbm.at[idsv.at[pl.ds(j * CH, CH)]],
                g0.at[pl.ds(j * CH, CH)], sem))
            copies.append(pltpu.async_copy(
                p1_hbm.at[idsv.at[pl.ds(j * CH, CH)]],
                g1.at[pl.ds(j * CH, CH)], sem))
        for c in copies:
            c.wait()

        pltpu.sync_copy(g0, out_hbm.at[0, pl.ds(base, bpw)])
        pltpu.sync_copy(g1, out_hbm.at[1, pl.ds(base, bpw)])

    return gather_k(p0, p1, idsT)


def kernel(ids, token_emb, fc_w, fc_b):
    B = ids.shape[0]
    V = token_emb.shape[0]
    idsT = ids.astype(jnp.int32).T
    token_embT = token_emb.T
    out2 = _router(B, V, idsT, token_embT, fc_w, fc_b.reshape(2, 1))
    return out2.T
